# baseline (speedup 1.0000x reference)
import functools

import jax
import jax.numpy as jnp
from jax import lax
from jax.experimental import pallas as pl
from jax.experimental.pallas import tpu as pltpu
from jax.experimental.pallas import tpu_sc as plsc

NC, NS, L = 2, 16, 16
NW = NC * NS
B, R, C = 2, 1000, 96
RPAD = 1024
NBOX = B * RPAD
BOX_PER_W = NBOX // NW
PER_BATCH = 21824
CB = C // L
PW = 192


def _unpack2(vi):
    a = plsc.bitcast(vi << 16, jnp.float32)
    b = plsc.bitcast(vi, jnp.float32)
    return a, b


def _body(table, props, out, boxes_v, idx_v, rows_v, fvecs, out_v, sem,
          osem0, osem1):
    wid = lax.axis_index("s") * NC + lax.axis_index("c")
    base_box = wid * BOX_PER_W
    pltpu.sync_copy(props.at[pl.ds(base_box * 4, BOX_PER_W * 4)],
                    boxes_v.at[pl.ds(0, BOX_PER_W * 4)])
    lane = lax.iota(jnp.int32, 16)
    lanef = lane.astype(jnp.float32)

    def box_body(i, carry):
        g = base_box + i
        par = lax.rem(i, 2)
        x1 = boxes_v[pl.ds(i * 4, 16)][0]
        y1 = boxes_v[pl.ds(i * 4 + 1, 16)][0]
        x2 = boxes_v[pl.ds(i * 4 + 2, 16)][0]
        y2 = boxes_v[pl.ds(i * 4 + 3, 16)][0]
        bw = jnp.maximum(x2 - x1, 1.0)
        bh = jnp.maximum(y2 - y1, 1.0)
        area = bw * bh
        ge3 = area >= 12544.0
        ge4 = area >= 50176.0
        ge5 = area >= 200704.0
        ge6 = area >= 802816.0
        scale = jnp.where(ge6, 0.015625,
                jnp.where(ge5, 0.03125,
                jnp.where(ge4, 0.0625,
                jnp.where(ge3, 0.125, 0.25))))
        wl = jnp.where(ge6, 8, jnp.where(ge5, 16, jnp.where(ge4, 32,
             jnp.where(ge3, 64, 128)))).astype(jnp.int32)
        lbase = jnp.where(ge6, 21760, jnp.where(ge5, 21504,
                jnp.where(ge4, 20480, jnp.where(ge3, 16384, 0)))).astype(jnp.int32)
        base = lbase + jnp.where(g >= RPAD, PER_BATCH, 0).astype(jnp.int32)
        wf = wl.astype(jnp.float32)

        x1s = x1 * scale
        y1s = y1 * scale
        x2s = x2 * scale
        y2s = y2 * scale
        bin_w = jnp.maximum(x2s - x1s, 1.0) * (1.0 / 7.0)
        bin_h = jnp.maximum(y2s - y1s, 1.0) * (1.0 / 7.0)

        ys = y1s + (0.5 * lanef + 0.25) * bin_h
        xs = x1s + (0.5 * lanef + 0.25) * bin_w
        vy = jnp.where((ys > -1.0) & (ys < wf), 1.0, 0.0)
        vx = jnp.where((xs > -1.0) & (xs < wf), 1.0, 0.0)
        yc = jnp.clip(ys, 0.0, wf - 1.0)
        xc = jnp.clip(xs, 0.0, wf - 1.0)
        y0i = yc.astype(jnp.int32)
        x0i = xc.astype(jnp.int32)
        ly = yc - y0i.astype(jnp.float32)
        lx = xc - x0i.astype(jnp.float32)
        at_xe = x0i == (wl - 1)
        hxv = jnp.where(at_xe, vx, (1.0 - lx) * vx)
        lxv = jnp.where(at_xe, 0.0, lx * vx)
        at_ye = y0i == (wl - 1)
        hyv = jnp.where(at_ye, vy, (1.0 - ly) * vy)
        lyv = jnp.where(at_ye, 0.0, ly * vy)
        rt = base + y0i * wl
        fvecs[pl.ds(0, 16)] = hyv
        fvecs[pl.ds(16, 16)] = lyv

        for y in range(14):
            part = y // 7
            off = (y % 7) * 16
            idx_v[part, pl.ds(off, 16)] = x0i + rt[y]

        h0 = pltpu.async_copy(table.at[idx_v.at[0]],
                              rows_v.at[pl.ds(0, 112)], sem)
        h1 = pltpu.async_copy(table.at[idx_v.at[1]],
                              rows_v.at[pl.ds(112, 112)], sem)
        h0.wait()
        h1.wait()

        bidx = jnp.where(g >= RPAD, 1, 0).astype(jnp.int32)
        r = g - bidx * RPAD
        prev_fired = (i >= 2) & (r - 2 >= 0) & (r - 2 < R)

        @pl.when(prev_fired & (par == 0))
        def _():
            pltpu.make_async_copy(out_v.at[pl.ds(0, 4704)], out.at[0],
                                  osem0).wait()

        @pl.when(prev_fired & (par == 1))
        def _():
            pltpu.make_async_copy(out_v.at[pl.ds(4704, 4704)], out.at[0],
                                  osem1).wait()

        def chunk_body(cc, carry2):
            hy0 = fvecs[pl.ds(2 * cc, 16)][0]
            ly0 = fvecs[pl.ds(16 + 2 * cc, 16)][0]
            hy1 = fvecs[pl.ds(2 * cc + 1, 16)][0]
            ly1 = fvecs[pl.ds(16 + 2 * cc + 1, 16)][0]
            hys = (hy0, hy1)
            lys = (ly0, ly1)
            for ox in range(7):
                acc = [jnp.zeros((16,), jnp.float32) for _ in range(CB)]
                for sy in range(2):
                    hy_s = hys[sy]
                    ly_s = lys[sy]
                    for sx in range(2):
                        xj = 2 * ox + sx
                        hx_s = hxv[xj]
                        lx_s = lxv[xj]
                        w00 = hy_s * hx_s
                        w01 = hy_s * lx_s
                        w10 = ly_s * hx_s
                        w11 = ly_s * lx_s
                        pr = (2 * cc + sy) * 16 + xj
                        for gch in range(3):
                            t0a, t0b = _unpack2(
                                rows_v[pr, pl.ds(gch * 16, 16)])
                            t1a, t1b = _unpack2(
                                rows_v[pr, pl.ds(48 + gch * 16, 16)])
                            b0a, b0b = _unpack2(
                                rows_v[pr, pl.ds(96 + gch * 16, 16)])
                            b1a, b1b = _unpack2(
                                rows_v[pr, pl.ds(144 + gch * 16, 16)])
                            acc[2 * gch] = (acc[2 * gch]
                                            + w00 * t0a + w01 * t1a
                                            + w10 * b0a + w11 * b1a)
                            acc[2 * gch + 1] = (acc[2 * gch + 1]
                                                + w00 * t0b + w01 * t1b
                                                + w10 * b0b + w11 * b1b)
                obase = par * 4704 + (cc * 7 + ox) * 96
                for k in range(CB):
                    out_v[pl.ds(obase + k * 16, 16)] = acc[k] * 0.25
            return carry2

        lax.fori_loop(0, 7, chunk_body, 0, unroll=False)

        orow = bidx * R + r

        @pl.when((r < R) & (par == 0))
        def _():
            pltpu.async_copy(out_v.at[pl.ds(0, 4704)], out.at[orow], osem0)

        @pl.when((r < R) & (par == 1))
        def _():
            pltpu.async_copy(out_v.at[pl.ds(4704, 4704)], out.at[orow],
                             osem1)

        return carry

    lax.fori_loop(0, BOX_PER_W, box_body, 0, unroll=False)
    g62 = base_box + 62
    r62 = g62 - jnp.where(g62 >= RPAD, RPAD, 0)
    g63 = base_box + 63
    r63 = g63 - jnp.where(g63 >= RPAD, RPAD, 0)

    @pl.when(r62 < R)
    def _():
        pltpu.make_async_copy(out_v.at[pl.ds(0, 4704)], out.at[0],
                              osem0).wait()

    @pl.when(r63 < R)
    def _():
        pltpu.make_async_copy(out_v.at[pl.ds(4704, 4704)], out.at[0],
                              osem1).wait()


@functools.partial(
    pl.kernel,
    mesh=plsc.VectorSubcoreMesh(core_axis_name="c", subcore_axis_name="s"),
    out_type=jax.ShapeDtypeStruct((B * R, 7 * 7 * C), jnp.float32),
    scratch_types=[
        pltpu.VMEM((BOX_PER_W * 4 + 16,), jnp.float32),
        pltpu.VMEM((2, 112), jnp.int32),
        pltpu.VMEM((224, PW), jnp.int32),
        pltpu.VMEM((48,), jnp.float32),
        pltpu.VMEM((2 * 7 * 7 * C,), jnp.float32),
        pltpu.SemaphoreType.DMA,
        pltpu.SemaphoreType.DMA,
        pltpu.SemaphoreType.DMA,
    ],
    compiler_params=pltpu.CompilerParams(use_tc_tiling_on_sc=False,
                                         needs_layout_passes=False),
)
def _roi_kernel(table, props, out, boxes_v, idx_v, rows_v, fvecs, out_v, sem,
                osem0, osem1):
    _body(table, props, out, boxes_v, idx_v, rows_v, fvecs, out_v, sem,
          osem0, osem1)


def kernel(p2, p3, p4, p5, p6, proposals):
    def prep(p):
        x = p.astype(jnp.bfloat16).reshape(B, p.shape[1], p.shape[2], 3, 2, 16)
        x = x.transpose(0, 1, 2, 3, 5, 4)
        xi = lax.bitcast_convert_type(x, jnp.int32).reshape(
            B, p.shape[1], p.shape[2], 48)
        xr = jnp.concatenate([xi[:, :, 1:], xi[:, :, -1:]], axis=2)
        xd = jnp.concatenate([xi[:, 1:], xi[:, -1:]], axis=1)
        xdr = jnp.concatenate([xd[:, :, 1:], xd[:, :, -1:]], axis=2)
        patch = jnp.concatenate([xi, xr, xd, xdr], axis=-1)
        return patch.reshape(B, -1, PW)

    table = jnp.concatenate([prep(p) for p in (p2, p3, p4, p5, p6)],
                            axis=1).reshape(B * PER_BATCH, PW)
    props = jnp.zeros((B, RPAD, 4), jnp.float32).at[:, :R].set(proposals)
    props = props.reshape(NBOX * 4)
    out = _roi_kernel(table, props)
    return out.reshape(B, R, 7, 7, C)


# speedup vs baseline: 1.2550x; 1.2550x over previous
import functools

import jax
import jax.numpy as jnp
from jax import lax
from jax.experimental import pallas as pl
from jax.experimental.pallas import tpu as pltpu
from jax.experimental.pallas import tpu_sc as plsc

NC, NS, L = 2, 16, 16
NW = NC * NS
B, R, C = 2, 1000, 96
RPAD = 1024
NBOX = B * RPAD
BOX_PER_W = NBOX // NW
PER_BATCH = 21824
CB = C // L
PW = 192


def _unpack2(vi):
    a = plsc.bitcast(vi << 16, jnp.float32)
    b = plsc.bitcast(vi, jnp.float32)
    return a, b


def _body(table, props, out, boxes_v, idx_v, rows_v, fvecs, out_v, sem,
          osem0, osem1):
    wid = lax.axis_index("s") * NC + lax.axis_index("c")
    base_box = wid * BOX_PER_W
    pltpu.sync_copy(props.at[pl.ds(base_box * 4, BOX_PER_W * 4)],
                    boxes_v.at[pl.ds(0, BOX_PER_W * 4)])
    lane = lax.iota(jnp.int32, 16)
    lanef = lane.astype(jnp.float32)
    lane2 = lane * 2

    def box_body(i, carry):
        g = base_box + i
        par = lax.rem(i, 2)
        x1 = boxes_v[pl.ds(i * 4, 16)][0]
        y1 = boxes_v[pl.ds(i * 4 + 1, 16)][0]
        x2 = boxes_v[pl.ds(i * 4 + 2, 16)][0]
        y2 = boxes_v[pl.ds(i * 4 + 3, 16)][0]
        bw = jnp.maximum(x2 - x1, 1.0)
        bh = jnp.maximum(y2 - y1, 1.0)
        area = bw * bh
        ge3 = area >= 12544.0
        ge4 = area >= 50176.0
        ge5 = area >= 200704.0
        ge6 = area >= 802816.0
        scale = jnp.where(ge6, 0.015625,
                jnp.where(ge5, 0.03125,
                jnp.where(ge4, 0.0625,
                jnp.where(ge3, 0.125, 0.25))))
        wl = jnp.where(ge6, 8, jnp.where(ge5, 16, jnp.where(ge4, 32,
             jnp.where(ge3, 64, 128)))).astype(jnp.int32)
        lbase = jnp.where(ge6, 21760, jnp.where(ge5, 21504,
                jnp.where(ge4, 20480, jnp.where(ge3, 16384, 0)))).astype(jnp.int32)
        base = lbase + jnp.where(g >= RPAD, PER_BATCH, 0).astype(jnp.int32)
        wf = wl.astype(jnp.float32)

        x1s = x1 * scale
        y1s = y1 * scale
        x2s = x2 * scale
        y2s = y2 * scale
        bin_w = jnp.maximum(x2s - x1s, 1.0) * (1.0 / 7.0)
        bin_h = jnp.maximum(y2s - y1s, 1.0) * (1.0 / 7.0)

        ys = y1s + (0.5 * lanef + 0.25) * bin_h
        xs = x1s + (0.5 * lanef + 0.25) * bin_w
        vy = jnp.where((ys > -1.0) & (ys < wf), 1.0, 0.0)
        vx = jnp.where((xs > -1.0) & (xs < wf), 1.0, 0.0)
        yc = jnp.clip(ys, 0.0, wf - 1.0)
        xc = jnp.clip(xs, 0.0, wf - 1.0)
        y0i = yc.astype(jnp.int32)
        x0i = xc.astype(jnp.int32)
        ly = yc - y0i.astype(jnp.float32)
        lx = xc - x0i.astype(jnp.float32)
        at_xe = x0i == (wl - 1)
        hxv = jnp.where(at_xe, vx, (1.0 - lx) * vx)
        lxv = jnp.where(at_xe, 0.0, lx * vx)
        hyv = (1.0 - ly) * vy
        lyv = ly * vy
        y1i = jnp.minimum(y0i + 1, wl - 1)
        rt = base + y0i * wl
        rb = base + y1i * wl
        fvecs[pl.ds(0, 16)] = hyv
        fvecs[pl.ds(16, 16)] = lyv

        for y in range(14):
            for h in range(2):
                b = 2 * y + h
                part = b // 7
                off = (b % 7) * 16
                src = rt[y] if h == 0 else rb[y]
                idx_v[part, pl.ds(off, 16)] = x0i + src

        handles = [
            pltpu.async_copy(table.at[idx_v.at[p]],
                             rows_v.at[pl.ds(p * 112, 112)], sem)
            for p in range(4)
        ]
        for hh in handles:
            hh.wait()

        bidx = jnp.where(g >= RPAD, 1, 0).astype(jnp.int32)
        r = g - bidx * RPAD
        prev_fired = (i >= 2) & (r - 2 >= 0) & (r - 2 < R)

        @pl.when(prev_fired & (par == 0))
        def _():
            pltpu.make_async_copy(out_v.at[pl.ds(0, 4704)], out.at[0],
                                  osem0).wait()

        @pl.when(prev_fired & (par == 1))
        def _():
            pltpu.make_async_copy(out_v.at[pl.ds(4704, 4704)], out.at[0],
                                  osem1).wait()

        def chunk_body(cc, carry2):
            hy0 = fvecs[pl.ds(2 * cc, 16)][0]
            ly0 = fvecs[pl.ds(16 + 2 * cc, 16)][0]
            hy1 = fvecs[pl.ds(2 * cc + 1, 16)][0]
            ly1 = fvecs[pl.ds(16 + 2 * cc + 1, 16)][0]
            hys = (hy0, hy1)
            lys = (ly0, ly1)
            for ox in range(7):
                acc = [jnp.zeros((16,), jnp.float32) for _ in range(CB)]
                for sy in range(2):
                    hy_s = hys[sy]
                    ly_s = lys[sy]
                    for sx in range(2):
                        xj = 2 * ox + sx
                        hx_s = hxv[xj]
                        lx_s = lxv[xj]
                        w00 = hy_s * hx_s
                        w01 = hy_s * lx_s
                        w10 = ly_s * hx_s
                        w11 = ly_s * lx_s
                        prt = (4 * cc + 2 * sy) * 16 + xj
                        prb = prt + 16
                        for gch in range(3):
                            t0a, t0b = _unpack2(
                                rows_v[prt, pl.ds(gch * 16, 16)])
                            t1a, t1b = _unpack2(
                                rows_v[prt, pl.ds(48 + gch * 16, 16)])
                            b0a, b0b = _unpack2(
                                rows_v[prb, pl.ds(gch * 16, 16)])
                            b1a, b1b = _unpack2(
                                rows_v[prb, pl.ds(48 + gch * 16, 16)])
                            acc[2 * gch] = (acc[2 * gch]
                                            + w00 * t0a + w01 * t1a
                                            + w10 * b0a + w11 * b1a)
                            acc[2 * gch + 1] = (acc[2 * gch + 1]
                                                + w00 * t0b + w01 * t1b
                                                + w10 * b0b + w11 * b1b)
                obase = par * 4704 + (cc * 7 + ox) * 96
                for gch in range(3):
                    sbase = obase + 32 * gch
                    plsc.store_scatter(out_v, [lane2 + sbase],
                                       acc[2 * gch] * 0.25)
                    plsc.store_scatter(out_v, [lane2 + (sbase + 1)],
                                       acc[2 * gch + 1] * 0.25)
            return carry2

        lax.fori_loop(0, 7, chunk_body, 0, unroll=False)

        orow = bidx * R + r

        @pl.when((r < R) & (par == 0))
        def _():
            pltpu.async_copy(out_v.at[pl.ds(0, 4704)], out.at[orow], osem0)

        @pl.when((r < R) & (par == 1))
        def _():
            pltpu.async_copy(out_v.at[pl.ds(4704, 4704)], out.at[orow],
                             osem1)

        return carry

    lax.fori_loop(0, BOX_PER_W, box_body, 0, unroll=False)
    g62 = base_box + 62
    r62 = g62 - jnp.where(g62 >= RPAD, RPAD, 0)
    g63 = base_box + 63
    r63 = g63 - jnp.where(g63 >= RPAD, RPAD, 0)

    @pl.when(r62 < R)
    def _():
        pltpu.make_async_copy(out_v.at[pl.ds(0, 4704)], out.at[0],
                              osem0).wait()

    @pl.when(r63 < R)
    def _():
        pltpu.make_async_copy(out_v.at[pl.ds(4704, 4704)], out.at[0],
                              osem1).wait()


@functools.partial(
    pl.kernel,
    mesh=plsc.VectorSubcoreMesh(core_axis_name="c", subcore_axis_name="s"),
    out_type=jax.ShapeDtypeStruct((B * R, 7 * 7 * C), jnp.float32),
    scratch_types=[
        pltpu.VMEM((BOX_PER_W * 4 + 16,), jnp.float32),
        pltpu.VMEM((4, 112), jnp.int32),
        pltpu.VMEM((448, 96), jnp.int32),
        pltpu.VMEM((48,), jnp.float32),
        pltpu.VMEM((2 * 7 * 7 * C,), jnp.float32),
        pltpu.SemaphoreType.DMA,
        pltpu.SemaphoreType.DMA,
        pltpu.SemaphoreType.DMA,
    ],
    compiler_params=pltpu.CompilerParams(use_tc_tiling_on_sc=False,
                                         needs_layout_passes=False),
)
def _roi_kernel(table, props, out, boxes_v, idx_v, rows_v, fvecs, out_v, sem,
                osem0, osem1):
    _body(table, props, out, boxes_v, idx_v, rows_v, fvecs, out_v, sem,
          osem0, osem1)


def kernel(p2, p3, p4, p5, p6, proposals):
    def prep(p):
        x = p.astype(jnp.bfloat16).reshape(B, -1, 48, 2)
        return lax.bitcast_convert_type(x, jnp.int32)

    t1 = jnp.concatenate([prep(p) for p in (p2, p3, p4, p5, p6)],
                         axis=1).reshape(B * PER_BATCH, 48)
    tpad = jnp.concatenate([t1, jnp.zeros((1, 48), jnp.int32)], axis=0)
    table = jnp.concatenate([tpad[:-1], tpad[1:]], axis=1)
    props = jnp.zeros((B, RPAD, 4), jnp.float32).at[:, :R].set(proposals)
    props = props.reshape(NBOX * 4)
    out = _roi_kernel(table, props)
    return out.reshape(B, R, 7, 7, C)
